# baseline (speedup 1.0000x reference)
import functools

import jax
import jax.numpy as jnp
from jax import lax
from jax.experimental import pallas as pl
from jax.experimental.pallas import tpu as pltpu
from jax.experimental.pallas import tpu_sc as plsc

N = 50000
D = 64
HW = D // 2
E = 800000
GAMMA = 40.0
NC = 2
NS = 16
NW = NC * NS
SUB = 128

CHA = 256
EPTA = 25088
NCHA = EPTA // CHA
PADE = EPTA * NW
PAD_EDGES = float(PADE - E)

CHB = 512
EPTB = 50176
NCHB = EPTB // CHB

NPAD = 50176
RPT = NPAD // NS
ZC = 112
TP = N + 8

_mesh = plsc.VectorSubcoreMesh(core_axis_name="c", subcore_axis_name="s")
_params = pltpu.CompilerParams(use_tc_tiling_on_sc=False)


def _permute(v, perm):
    dnums = lax.GatherDimensionNumbers(
        offset_dims=(), collapsed_slice_dims=(0,), start_index_map=(0,))
    return lax.gather(v, perm[:, None], dnums, slice_sizes=(1,),
                      mode=lax.GatherScatterMode.PROMISE_IN_BOUNDS)


@functools.partial(
    pl.kernel,
    mesh=_mesh,
    compiler_params=_params,
    out_type=(
        jax.ShapeDtypeStruct((PADE,), jnp.float32),
        jax.ShapeDtypeStruct((NW, 16), jnp.float32),
    ),
    scratch_types=[
        pltpu.VMEM((2, 2, CHA), jnp.int32),
        pltpu.VMEM((2, CHA, D), jnp.float32),
        pltpu.VMEM((2, CHA, D), jnp.float32),
        pltpu.VMEM((CHA,), jnp.float32),
        pltpu.VMEM((16,), jnp.float32),
        pltpu.VMEM((16, 16), jnp.float32),
        pltpu.SemaphoreType.DMA,
        pltpu.SemaphoreType.DMA,
        pltpu.SemaphoreType.DMA,
        pltpu.SemaphoreType.DMA,
    ],
)
def _score_pass(idx2, dst_tab, src_tab, w_out, z_out,
                ib, rows_d, rows_s, wbuf, z_acc, ohm, sem1, sem2, sem3, sem4):
    cid = lax.axis_index("c")
    sid = lax.axis_index("s")
    wid = sid * NC + cid

    z_acc[...] = jnp.zeros((16,), jnp.float32)
    lanef0 = lax.iota(jnp.int32, 16).astype(jnp.float32)
    for i in range(16):
        dd0 = lanef0 - float(i)
        ohm[i, pl.ds(0, 16)] = jnp.maximum(0.0, 1.0 - dd0 * dd0)

    lane = lax.iota(jnp.int32, 16)
    sems = ((sem1, sem2), (sem3, sem4))

    def _chunk(j, carry):
        cps = []
        for h in range(2):
            pltpu.sync_copy(idx2.at[wid * NCHA + 2 * j + h], ib.at[h])
            cps.append((
                pltpu.async_copy(dst_tab.at[ib.at[h, 0]], rows_d.at[h], sems[h][0]),
                pltpu.async_copy(src_tab.at[ib.at[h, 1]], rows_s.at[h], sems[h][1]),
            ))
        for h in range(2):
            base = wid * EPTA + (2 * j + h) * CHA
            cps[h][0].wait()
            cps[h][1].wait()

            def _sub(s, carry2, h=h):
                es = s * SUB
                zsum = jnp.zeros((16,), jnp.float32)
                for g in range(SUB // 16):
                    tg = jnp.zeros((16,), jnp.float32)
                    for i in range(16):
                        e = es + g * 16 + i
                        d0 = rows_d[h, e, pl.ds(0, 32)]
                        d1 = rows_d[h, e, pl.ds(32, 32)]
                        s0 = rows_s[h, e, pl.ds(0, 32)]
                        s1 = rows_s[h, e, pl.ds(32, 32)]
                        p = d0 * s0 + d1 * s1
                        t = p[:16] + p[16:]
                        for hh in (8, 4, 2, 1):
                            t = t + _permute(t, jnp.bitwise_xor(lane, hh))
                        tg = tg + t * ohm[i, pl.ds(0, 16)]
                    wg = jnp.exp(tg)
                    zsum = zsum + wg
                    wbuf[pl.ds(es + g * 16, 16)] = wg
                z_acc[...] = z_acc[...] + zsum
                return carry2
            lax.fori_loop(0, CHA // SUB, _sub, 0)

            pltpu.sync_copy(wbuf, w_out.at[pl.ds(base, CHA)])
        return carry
    lax.fori_loop(0, NCHA // 2, _chunk, 0)

    pltpu.sync_copy(z_acc, z_out.at[wid])


@functools.partial(
    pl.kernel,
    mesh=_mesh,
    compiler_params=_params,
    out_type=jax.ShapeDtypeStruct((NC, NPAD, HW), jnp.float32),
    scratch_types=[
        pltpu.VMEM((2, CHB), jnp.int32),
        pltpu.VMEM((CHB,), jnp.float32),
        pltpu.VMEM((CHB, HW), jnp.float32),
        pltpu.VMEM_SHARED((NPAD, HW), jnp.float32),
        pltpu.SemaphoreType.DMA,
        pltpu.SemaphoreType.DMA,
    ],
)
def _scatter_pass(idx2, w_in, tab_l, tab_r, agg_out,
                  ib, wv, srows, agg, sem1, sem2):
    cid = lax.axis_index("c")
    sid = lax.axis_index("s")
    cid_is0 = cid == 0

    zf = jnp.zeros((16,), jnp.float32)

    def _zrow(r, c):
        srows[r, pl.ds(0, 16)] = zf
        srows[r, pl.ds(16, 16)] = zf
        return c
    lax.fori_loop(0, CHB, _zrow, 0)

    def _zagg(j, c):
        pltpu.sync_copy(srows.at[pl.ds(0, ZC)],
                        agg.at[pl.ds(sid * RPT + j * ZC, ZC)])
        return c
    lax.fori_loop(0, RPT // ZC, _zagg, 0)

    plsc.subcore_barrier()

    def _chunk(k, carry):
        base = sid * EPTB + k * CHB
        pltpu.sync_copy(idx2.at[sid * NCHB + k], ib)
        pltpu.sync_copy(w_in.at[pl.ds(base, CHB)], wv)

        @pl.when(cid_is0)
        def _():
            pltpu.async_copy(tab_l.at[ib.at[1]], srows, sem1).wait()

        @pl.when(jnp.logical_not(cid_is0))
        def _():
            pltpu.async_copy(tab_r.at[ib.at[1]], srows, sem2).wait()

        def _sub(s, carry2):
            es = s * SUB
            for g in range(SUB // 16):
                wg = wv[pl.ds(es + g * 16, 16)]
                for i in range(16):
                    e = es + g * 16 + i
                    w32 = jnp.broadcast_to(wg[i], (32,))
                    srows[e, pl.ds(0, 32)] = srows[e, pl.ds(0, 32)] * w32
            return carry2
        lax.fori_loop(0, CHB // SUB, _sub, 0)

        pltpu.sync_copy(srows, agg.at[ib.at[0]], add=True)
        return carry
    lax.fori_loop(0, NCHB, _chunk, 0)

    plsc.subcore_barrier()

    pltpu.sync_copy(agg.at[pl.ds(sid * RPT, RPT)],
                    agg_out.at[cid, pl.ds(sid * RPT, RPT)])


_UPD_BLK = 5000


def _update_body(dst_ref, agg_ref, z_ref, out_ref):
    zsum = jnp.sum(z_ref[...]) - PAD_EDGES
    c = 1.0 / ((GAMMA + 1.0) * zsum)
    out_ref[...] = dst_ref[...] + agg_ref[...] * c


@jax.jit
def _update(dst, agg, z):
    grid = N // _UPD_BLK
    return pl.pallas_call(
        _update_body,
        grid=(grid,),
        in_specs=[
            pl.BlockSpec((_UPD_BLK, D), lambda i: (i, 0)),
            pl.BlockSpec((_UPD_BLK, D), lambda i: (i, 0)),
            pl.BlockSpec((NW, 16), lambda i: (0, 0)),
        ],
        out_specs=pl.BlockSpec((_UPD_BLK, D), lambda i: (i, 0)),
        out_shape=jax.ShapeDtypeStruct((N, D), jnp.float32),
    )(dst, agg, z)


def _padtab(t):
    return jnp.concatenate([t, jnp.zeros((TP - N, D), jnp.float32)])


def _assemble_agg(agg):
    return jnp.concatenate([agg[0, :N], agg[1, :N]], axis=1)


def _pack_a(dst_i, src_i):
    d = dst_i.reshape(NW, NCHA, CHA)
    s = src_i.reshape(NW, NCHA, CHA)
    return jnp.stack([d, s], axis=2).reshape(NW * NCHA, 2, CHA)


def _pack_b(dst_i, src_i):
    d = dst_i.reshape(NS, NCHB, CHB)
    s = src_i.reshape(NS, NCHB, CHB)
    return jnp.stack([d, s], axis=2).reshape(NS * NCHB, 2, CHB)


def _pass(pka, pkb, dst_t, src_t):
    dst_p = _padtab(dst_t)
    src_p = _padtab(src_t)
    w, z = _score_pass(pka, dst_p, src_p)
    agg = _scatter_pass(pkb, w, src_p[:, :HW], src_p[:, HW:])
    return _update(dst_t, _assemble_agg(agg), z)


def kernel(edge_index, user_weight, item_weight):
    row = edge_index[0].astype(jnp.int32)
    col = edge_index[1].astype(jnp.int32)
    pad = jnp.full((PADE - E,), N, jnp.int32)
    rowp = jnp.concatenate([row, pad])
    colp = jnp.concatenate([col, pad])

    pka_rc = _pack_a(rowp, colp)
    pka_cr = _pack_a(colp, rowp)
    pkb_rc = _pack_b(rowp, colp)
    pkb_cr = _pack_b(colp, rowp)

    u = user_weight
    it = item_weight
    for _ in range(4):
        u = _pass(pka_rc, pkb_rc, u, it)
        it = _pass(pka_cr, pkb_cr, it, u)
    return u, it


# speedup vs baseline: 1.2160x; 1.2160x over previous
import functools

import jax
import jax.numpy as jnp
from jax import lax
from jax.experimental import pallas as pl
from jax.experimental.pallas import tpu as pltpu
from jax.experimental.pallas import tpu_sc as plsc

N = 50000
D = 64
HW = D // 2
E = 800000
GAMMA = 40.0
NC = 2
NS = 16
NW = NC * NS
SUB = 128

CHA = 512
EPTA = 25088
NCHA = EPTA // CHA
PADE = EPTA * NW
PAD_EDGES = float(PADE - E)

CHB = 512
EPTB = 50176
NCHB = EPTB // CHB

NPAD = 50176
RPT = NPAD // NS
ZC = 112
TP = N + 8

_mesh = plsc.VectorSubcoreMesh(core_axis_name="c", subcore_axis_name="s")
_params = pltpu.CompilerParams(use_tc_tiling_on_sc=False)


def _permute(v, perm):
    dnums = lax.GatherDimensionNumbers(
        offset_dims=(), collapsed_slice_dims=(0,), start_index_map=(0,))
    return lax.gather(v, perm[:, None], dnums, slice_sizes=(1,),
                      mode=lax.GatherScatterMode.PROMISE_IN_BOUNDS)


@functools.partial(
    pl.kernel,
    mesh=_mesh,
    compiler_params=_params,
    out_type=(
        jax.ShapeDtypeStruct((PADE,), jnp.float32),
        jax.ShapeDtypeStruct((NW, 16), jnp.float32),
    ),
    scratch_types=[
        pltpu.VMEM((2, CHA), jnp.int32),
        pltpu.VMEM((CHA, D), jnp.float32),
        pltpu.VMEM((CHA, D), jnp.float32),
        pltpu.VMEM((CHA,), jnp.float32),
        pltpu.VMEM((16,), jnp.float32),
        pltpu.VMEM((16, 16), jnp.float32),
        pltpu.SemaphoreType.DMA,
        pltpu.SemaphoreType.DMA,
    ],
)
def _score_pass(idx2, dst_tab, src_tab, w_out, z_out,
                ib, rows_d, rows_s, wbuf, z_acc, ohm, sem1, sem2):
    cid = lax.axis_index("c")
    sid = lax.axis_index("s")
    wid = sid * NC + cid

    z_acc[...] = jnp.zeros((16,), jnp.float32)
    lanef0 = lax.iota(jnp.int32, 16).astype(jnp.float32)
    for i in range(16):
        dd0 = lanef0 - float(i)
        ohm[i, pl.ds(0, 16)] = jnp.maximum(0.0, 1.0 - dd0 * dd0)

    def _chunk(k, carry):
        base = wid * EPTA + k * CHA
        pltpu.sync_copy(idx2.at[wid * NCHA + k], ib)
        cp1 = pltpu.async_copy(dst_tab.at[ib.at[0]], rows_d, sem1)
        cp2 = pltpu.async_copy(src_tab.at[ib.at[1]], rows_s, sem2)
        cp1.wait()
        cp2.wait()

        lane = lax.iota(jnp.int32, 16)

        def _sub(s, carry2):
            es = s * SUB
            zsum = jnp.zeros((16,), jnp.float32)
            for g in range(SUB // 16):
                tg = jnp.zeros((16,), jnp.float32)
                for i in range(16):
                    e = es + g * 16 + i
                    d0 = rows_d[e, pl.ds(0, 32)]
                    d1 = rows_d[e, pl.ds(32, 32)]
                    s0 = rows_s[e, pl.ds(0, 32)]
                    s1 = rows_s[e, pl.ds(32, 32)]
                    p = d0 * s0 + d1 * s1
                    t = p[:16] + p[16:]
                    for h in (8, 4, 2, 1):
                        t = t + _permute(t, jnp.bitwise_xor(lane, h))
                    tg = tg + t * ohm[i, pl.ds(0, 16)]
                wg = jnp.exp(tg)
                zsum = zsum + wg
                wbuf[pl.ds(es + g * 16, 16)] = wg
            z_acc[...] = z_acc[...] + zsum
            return carry2
        lax.fori_loop(0, CHA // SUB, _sub, 0)

        pltpu.sync_copy(wbuf, w_out.at[pl.ds(base, CHA)])
        return carry
    lax.fori_loop(0, NCHA, _chunk, 0)

    pltpu.sync_copy(z_acc, z_out.at[wid])


@functools.partial(
    pl.kernel,
    mesh=_mesh,
    compiler_params=_params,
    out_type=jax.ShapeDtypeStruct((NC, NPAD, HW), jnp.float32),
    scratch_types=[
        pltpu.VMEM((2, CHB), jnp.int32),
        pltpu.VMEM((CHB,), jnp.float32),
        pltpu.VMEM((CHB, HW), jnp.float32),
        pltpu.VMEM_SHARED((NPAD, HW), jnp.float32),
        pltpu.SemaphoreType.DMA,
        pltpu.SemaphoreType.DMA,
    ],
)
def _scatter_pass(idx2, w_in, tab_l, tab_r, agg_out,
                  ib, wv, srows, agg, sem1, sem2):
    cid = lax.axis_index("c")
    sid = lax.axis_index("s")
    cid_is0 = cid == 0

    zf = jnp.zeros((16,), jnp.float32)

    def _zrow(r, c):
        srows[r, pl.ds(0, 16)] = zf
        srows[r, pl.ds(16, 16)] = zf
        return c
    lax.fori_loop(0, CHB, _zrow, 0)

    def _zagg(j, c):
        pltpu.sync_copy(srows.at[pl.ds(0, ZC)],
                        agg.at[pl.ds(sid * RPT + j * ZC, ZC)])
        return c
    lax.fori_loop(0, RPT // ZC, _zagg, 0)

    plsc.subcore_barrier()

    def _chunk(k, carry):
        base = sid * EPTB + k * CHB
        pltpu.sync_copy(idx2.at[sid * NCHB + k], ib)
        pltpu.sync_copy(w_in.at[pl.ds(base, CHB)], wv)

        @pl.when(cid_is0)
        def _():
            pltpu.async_copy(tab_l.at[ib.at[1]], srows, sem1).wait()

        @pl.when(jnp.logical_not(cid_is0))
        def _():
            pltpu.async_copy(tab_r.at[ib.at[1]], srows, sem2).wait()

        def _sub(s, carry2):
            es = s * SUB
            for g in range(SUB // 16):
                wg = wv[pl.ds(es + g * 16, 16)]
                for i in range(16):
                    e = es + g * 16 + i
                    w32 = jnp.broadcast_to(wg[i], (32,))
                    srows[e, pl.ds(0, 32)] = srows[e, pl.ds(0, 32)] * w32
            return carry2
        lax.fori_loop(0, CHB // SUB, _sub, 0)

        pltpu.sync_copy(srows, agg.at[ib.at[0]], add=True)
        return carry
    lax.fori_loop(0, NCHB, _chunk, 0)

    plsc.subcore_barrier()

    pltpu.sync_copy(agg.at[pl.ds(sid * RPT, RPT)],
                    agg_out.at[cid, pl.ds(sid * RPT, RPT)])


_UPD_BLK = 5000


def _update_body(dst_ref, agg_ref, z_ref, out_ref):
    zsum = jnp.sum(z_ref[...]) - PAD_EDGES
    c = 1.0 / ((GAMMA + 1.0) * zsum)
    out_ref[...] = dst_ref[...] + agg_ref[...] * c


@jax.jit
def _update(dst, agg, z):
    grid = N // _UPD_BLK
    return pl.pallas_call(
        _update_body,
        grid=(grid,),
        in_specs=[
            pl.BlockSpec((_UPD_BLK, D), lambda i: (i, 0)),
            pl.BlockSpec((_UPD_BLK, D), lambda i: (i, 0)),
            pl.BlockSpec((NW, 16), lambda i: (0, 0)),
        ],
        out_specs=pl.BlockSpec((_UPD_BLK, D), lambda i: (i, 0)),
        out_shape=jax.ShapeDtypeStruct((N, D), jnp.float32),
    )(dst, agg, z)


def _padtab(t):
    return jnp.concatenate([t, jnp.zeros((TP - N, D), jnp.float32)])


def _assemble_agg(agg):
    return jnp.concatenate([agg[0, :N], agg[1, :N]], axis=1)


def _pack_a(dst_i, src_i):
    d = dst_i.reshape(NW, NCHA, CHA)
    s = src_i.reshape(NW, NCHA, CHA)
    return jnp.stack([d, s], axis=2).reshape(NW * NCHA, 2, CHA)


def _pack_b(dst_i, src_i):
    d = dst_i.reshape(NS, NCHB, CHB)
    s = src_i.reshape(NS, NCHB, CHB)
    return jnp.stack([d, s], axis=2).reshape(NS * NCHB, 2, CHB)


def _pass(pka, pkb, dst_t, src_t):
    dst_p = _padtab(dst_t)
    src_p = _padtab(src_t)
    w, z = _score_pass(pka, dst_p, src_p)
    agg = _scatter_pass(pkb, w, src_p[:, :HW], src_p[:, HW:])
    return _update(dst_t, _assemble_agg(agg), z)


def kernel(edge_index, user_weight, item_weight):
    row = edge_index[0].astype(jnp.int32)
    col = edge_index[1].astype(jnp.int32)
    pad = jnp.full((PADE - E,), N, jnp.int32)
    rowp = jnp.concatenate([row, pad])
    colp = jnp.concatenate([col, pad])

    pka_rc = _pack_a(rowp, colp)
    pka_cr = _pack_a(colp, rowp)
    pkb_rc = _pack_b(rowp, colp)
    pkb_cr = _pack_b(colp, rowp)

    u = user_weight
    it = item_weight
    for _ in range(4):
        u = _pass(pka_rc, pkb_rc, u, it)
        it = _pass(pka_cr, pkb_cr, it, u)
    return u, it
